# baseline (speedup 1.0000x reference)
import jax
import jax.numpy as jnp
from jax.experimental import pallas as pl
from jax.experimental.pallas import tpu as pltpu

_B = 64
_N = 32768
_K = 1000
_R = 16
_C = _B // 4


def _topk_mask(d, w):
    a = jnp.abs(d)
    bits = jax.lax.bitcast_convert_type(a, jnp.int32)
    one = jnp.int32(1)

    def kth_body(i, cur):
        cand = jnp.bitwise_or(cur, jnp.left_shift(one, 30 - i))
        cnt = jnp.sum((bits >= cand).astype(jnp.int32), axis=1,
                      keepdims=True)
        return jnp.where(cnt >= _K, cand, cur)

    kth = jax.lax.fori_loop(0, 31, kth_body, jnp.zeros((_R, 1), jnp.int32))

    gt = bits > kth
    need = _K - jnp.sum(gt.astype(jnp.int32), axis=1, keepdims=True)
    eq = bits == kth
    idx = jax.lax.broadcasted_iota(jnp.int32, bits.shape, 1)

    def cut_body(i, cur):
        cand = jnp.bitwise_or(cur, jnp.left_shift(one, 15 - i))
        cnt = jnp.sum((eq & (idx < cand)).astype(jnp.int32), axis=1,
                      keepdims=True)
        return jnp.where(cnt <= need, cand, cur)

    cut = jax.lax.fori_loop(0, 16, cut_body, jnp.zeros((_R, 1), jnp.int32))

    sel = gt | (eq & (idx < cut))
    return w + sel.astype(jnp.float32)


def _max1_kernel(cond_ref, d_hbm, w_hbm, o_hbm, d_s, w_s, o_s,
                 s0, s1, s2, s3):
    @pl.when(cond_ref[0] == 0)
    def _copy():
        sems = (s0, s1, s2, s3)
        copies = [
            pltpu.make_async_copy(w_hbm.at[pl.ds(c * _C, _C)],
                                  o_hbm.at[pl.ds(c * _C, _C)], sems[c])
            for c in range(4)
        ]
        for cp in copies:
            cp.start()
        for cp in copies:
            cp.wait()

    @pl.when(cond_ref[0] != 0)
    def _mask():
        def body(b, carry):
            rows = pl.ds(b * _R, _R)
            cp_d = pltpu.make_async_copy(d_hbm.at[rows], d_s, s0)
            cp_w = pltpu.make_async_copy(w_hbm.at[rows], w_s, s1)
            cp_d.start()
            cp_w.start()
            cp_d.wait()
            cp_w.wait()
            o_s[...] = _topk_mask(d_s[...], w_s[...])
            cp_o = pltpu.make_async_copy(o_s, o_hbm.at[rows], s2)
            cp_o.start()
            cp_o.wait()
            return carry

        jax.lax.fori_loop(0, _B // _R, body, 0)


def kernel(difference, weight, epoch, iteration):
    del iteration
    epoch = jnp.asarray(epoch, jnp.int32)
    cond = ((epoch > 1000) & (epoch < 18000)
            & (epoch % 200 == 0)).astype(jnp.int32).reshape(1)

    out = pl.pallas_call(
        _max1_kernel,
        in_specs=[
            pl.BlockSpec(memory_space=pltpu.SMEM),
            pl.BlockSpec(memory_space=pl.ANY),
            pl.BlockSpec(memory_space=pl.ANY),
        ],
        out_specs=pl.BlockSpec(memory_space=pl.ANY),
        out_shape=jax.ShapeDtypeStruct((_B, _N), jnp.float32),
        scratch_shapes=[
            pltpu.VMEM((_R, _N), jnp.float32),
            pltpu.VMEM((_R, _N), jnp.float32),
            pltpu.VMEM((_R, _N), jnp.float32),
            pltpu.SemaphoreType.DMA,
            pltpu.SemaphoreType.DMA,
            pltpu.SemaphoreType.DMA,
            pltpu.SemaphoreType.DMA,
        ],
    )(cond, difference, weight)
    return out


# speedup vs baseline: 33.0704x; 33.0704x over previous
import jax
import jax.numpy as jnp
from jax.experimental import pallas as pl
from jax.experimental.pallas import tpu as pltpu

_B = 64
_N = 32768
_K = 1000
_R = 32


def _max1_kernel(cond_ref, d_ref, w_ref, o_ref):
    @pl.when(cond_ref[0] == 0)
    def _copy():
        o_ref[...] = w_ref[...]

    @pl.when(cond_ref[0] != 0)
    def _topk_mask():
        a = jnp.abs(d_ref[...])
        bits = jax.lax.bitcast_convert_type(a, jnp.int32)
        one = jnp.int32(1)

        def kth_body(i, cur):
            cand = jnp.bitwise_or(cur, jnp.left_shift(one, 30 - i))
            cnt = jnp.sum((bits >= cand).astype(jnp.int32), axis=1,
                          keepdims=True)
            return jnp.where(cnt >= _K, cand, cur)

        kth = jax.lax.fori_loop(0, 31, kth_body,
                                jnp.zeros((_R, 1), jnp.int32))

        gt = bits > kth
        need = _K - jnp.sum(gt.astype(jnp.int32), axis=1, keepdims=True)
        eq = bits == kth
        idx = jax.lax.broadcasted_iota(jnp.int32, bits.shape, 1)

        def cut_body(i, cur):
            cand = jnp.bitwise_or(cur, jnp.left_shift(one, 15 - i))
            cnt = jnp.sum((eq & (idx < cand)).astype(jnp.int32), axis=1,
                          keepdims=True)
            return jnp.where(cnt <= need, cand, cur)

        cut = jax.lax.fori_loop(0, 16, cut_body,
                                jnp.zeros((_R, 1), jnp.int32))

        sel = gt | (eq & (idx < cut))
        o_ref[...] = w_ref[...] + sel.astype(jnp.float32)


def kernel(difference, weight, epoch, iteration):
    del iteration
    epoch = jnp.asarray(epoch, jnp.int32)
    cond = ((epoch > 1000) & (epoch < 18000)
            & (epoch % 200 == 0)).astype(jnp.int32).reshape(1)

    grid = _B // _R
    out = pl.pallas_call(
        _max1_kernel,
        grid_spec=pltpu.PrefetchScalarGridSpec(
            num_scalar_prefetch=1,
            grid=(grid,),
            in_specs=[
                pl.BlockSpec(
                    (_R, _N),
                    lambda i, cond_ref: (
                        jnp.where(cond_ref[0] != 0, i, 0), 0)),
                pl.BlockSpec((_R, _N), lambda i, cond_ref: (i, 0)),
            ],
            out_specs=pl.BlockSpec((_R, _N), lambda i, cond_ref: (i, 0)),
        ),
        out_shape=jax.ShapeDtypeStruct((_B, _N), jnp.float32),
    )(cond, difference, weight)
    return out


# speedup vs baseline: 38.8684x; 1.1753x over previous
import jax
import jax.numpy as jnp
from jax.experimental import pallas as pl
from jax.experimental.pallas import tpu as pltpu

_B = 64
_N = 32768
_K = 1000
_R = 32


def _max1_kernel(cond_ref, d_hbm, w_ref, o_ref, d_s, sem):
    @pl.when(cond_ref[0] == 0)
    def _copy():
        o_ref[...] = w_ref[...]

    @pl.when(cond_ref[0] != 0)
    def _topk_mask():
        i = pl.program_id(0)
        cp = pltpu.make_async_copy(d_hbm.at[pl.ds(i * _R, _R)], d_s, sem)
        cp.start()
        cp.wait()
        a = jnp.abs(d_s[...])
        bits = jax.lax.bitcast_convert_type(a, jnp.int32)
        one = jnp.int32(1)

        def kth_body(i, cur):
            cand = jnp.bitwise_or(cur, jnp.left_shift(one, 30 - i))
            cnt = jnp.sum((bits >= cand).astype(jnp.int32), axis=1,
                          keepdims=True)
            return jnp.where(cnt >= _K, cand, cur)

        kth = jax.lax.fori_loop(0, 31, kth_body,
                                jnp.zeros((_R, 1), jnp.int32))

        gt = bits > kth
        need = _K - jnp.sum(gt.astype(jnp.int32), axis=1, keepdims=True)
        eq = bits == kth
        idx = jax.lax.broadcasted_iota(jnp.int32, bits.shape, 1)

        def cut_body(i, cur):
            cand = jnp.bitwise_or(cur, jnp.left_shift(one, 15 - i))
            cnt = jnp.sum((eq & (idx < cand)).astype(jnp.int32), axis=1,
                          keepdims=True)
            return jnp.where(cnt <= need, cand, cur)

        cut = jax.lax.fori_loop(0, 16, cut_body,
                                jnp.zeros((_R, 1), jnp.int32))

        sel = gt | (eq & (idx < cut))
        o_ref[...] = w_ref[...] + sel.astype(jnp.float32)


def kernel(difference, weight, epoch, iteration):
    del iteration
    epoch = jnp.asarray(epoch, jnp.int32)
    cond = ((epoch > 1000) & (epoch < 18000)
            & (epoch % 200 == 0)).astype(jnp.int32).reshape(1)

    grid = _B // _R
    out = pl.pallas_call(
        _max1_kernel,
        grid_spec=pltpu.PrefetchScalarGridSpec(
            num_scalar_prefetch=1,
            grid=(grid,),
            in_specs=[
                pl.BlockSpec(memory_space=pl.ANY),
                pl.BlockSpec((_R, _N), lambda i, cond_ref: (i, 0)),
            ],
            out_specs=pl.BlockSpec((_R, _N), lambda i, cond_ref: (i, 0)),
            scratch_shapes=[
                pltpu.VMEM((_R, _N), jnp.float32),
                pltpu.SemaphoreType.DMA,
            ],
        ),
        out_shape=jax.ShapeDtypeStruct((_B, _N), jnp.float32),
    )(cond, difference, weight)
    return out
